# baseline (speedup 1.0000x reference)
import jax
import jax.numpy as jnp
from jax import lax
from jax.experimental import pallas as pl
from jax.experimental.pallas import tpu as pltpu
from jax.experimental.pallas import tpu_sc as plsc

_RADIUS2 = 3.4 * 3.4
_NSAMPLE = 5
_B = 8
_N = 2048
_L = 16
_NC = 2
_NS = 16
_NW = _NC * _NS
_WPB = _NW // _B
_QPW = _N // _WPB
_GROUPS = _QPW // _L
_BLK = 16


def _ball_query_body(x_hbm, out_hbm, pts_v, out_v):
    c = lax.axis_index("c")
    s = lax.axis_index("s")
    wid = s * _NC + c
    b = wid // _WPB
    qoff = (wid % _WPB) * _QPW

    pltpu.sync_copy(x_hbm.at[pl.ds(b * (_N * 3), _N * 3)], pts_v)

    lanes = lax.iota(jnp.int32, _L)
    lanes3 = lanes + (lanes + lanes)
    czero = jnp.zeros((_L,), jnp.int32)
    cone = jnp.full((_L,), 1, jnp.int32)
    ctwo = jnp.full((_L,), 2, jnp.int32)
    zero = jnp.zeros((_L,), jnp.int32)

    def group(g, carry_none):
        rows = jnp.full((_L,), g * _L, jnp.int32) + lanes
        qidx = (jnp.full((_L,), qoff, jnp.int32) + rows) * 3
        qx = plsc.load_gather(pts_v, [qidx])
        qy = plsc.load_gather(pts_v, [qidx + 1])
        qz = plsc.load_gather(pts_v, [qidx + 2])

        def cond(carry):
            return carry[0]

        def body(carry):
            j = carry[1]
            st = carry[2:]
            cnt, i0, i1, i2, i3, i4 = st[:6]
            jv0 = jnp.full((_L,), j, dtype=jnp.int32)
            j3 = jv0 + (jv0 + jv0) + lanes3
            xs16 = plsc.load_gather(pts_v, [j3])
            ys16 = plsc.load_gather(pts_v, [j3 + 1])
            zs16 = plsc.load_gather(pts_v, [j3 + 2])
            for u in range(_BLK):
                uvec = jnp.full((_L,), u, jnp.int32)
                px = jnp.take(xs16, uvec)
                py = jnp.take(ys16, uvec)
                pz = jnp.take(zs16, uvec)
                jv = jv0 + u
                dx = qx - px
                dy = qy - py
                dz = qz - pz
                d2 = dx * dx + dy * dy + dz * dz
                m = d2 < _RADIUS2
                i0 = jnp.where(m & (cnt == 0), jv, i0)
                i1 = jnp.where(m & (cnt == 1), jv, i1)
                i2 = jnp.where(m & (cnt == 2), jv, i2)
                i3 = jnp.where(m & (cnt == 3), jv, i3)
                i4 = jnp.where(m & (cnt == 4), jv, i4)
                cnt = cnt + m.astype(jnp.int32)
            jn = j + _BLK
            popc = plsc.all_reduce_population_count(cnt >= _NSAMPLE)
            cont = jnp.logical_and(jn < _N, popc[0] < _L)
            return (cont, jn, cnt, i0, i1, i2, i3, i4)

        init = (jnp.bool_(True), jnp.int32(0), zero, zero, zero, zero, zero,
                zero)
        res = lax.while_loop(cond, body, init)
        cnt, i0, i1, i2, i3, i4 = (res[2], res[3], res[4], res[5], res[6],
                                   res[7])

        o1 = jnp.where(cnt > 1, i1, i0)
        o2 = jnp.where(cnt > 2, i2, i0)
        o3 = jnp.where(cnt > 3, i3, i0)
        o4 = jnp.where(cnt > 4, i4, i0)
        plsc.store_scatter(out_v, [rows, czero], i0)
        plsc.store_scatter(out_v, [rows, cone], o1)
        plsc.store_scatter(out_v, [rows, ctwo], o2)
        plsc.store_scatter(out_v, [rows, ctwo + 1], o3)
        plsc.store_scatter(out_v, [rows, ctwo + 2], o4)
        return carry_none

    lax.fori_loop(0, _GROUPS, group, 0)

    pltpu.sync_copy(out_v, out_hbm.at[b].at[pl.ds(qoff, _QPW)])


def kernel(x):
    mesh = plsc.VectorSubcoreMesh(core_axis_name="c", subcore_axis_name="s")
    return pl.kernel(
        _ball_query_body,
        out_type=jax.ShapeDtypeStruct((_B, _N, _NSAMPLE), jnp.int32),
        mesh=mesh,
        compiler_params=pltpu.CompilerParams(needs_layout_passes=False),
        scratch_types=[
            pltpu.VMEM((_N * 3,), jnp.float32),
            pltpu.VMEM((_QPW, _NSAMPLE), jnp.int32),
        ],
    )(x.reshape(-1))


# speedup vs baseline: 1.1211x; 1.1211x over previous
import jax
import jax.numpy as jnp
from jax import lax
from jax.experimental import pallas as pl
from jax.experimental.pallas import tpu as pltpu
from jax.experimental.pallas import tpu_sc as plsc

_RADIUS2 = 3.4 * 3.4
_NSAMPLE = 5
_B = 8
_N = 2048
_L = 16
_NC = 2
_NS = 16
_NW = _NC * _NS
_WPB = _NW // _B
_QPW = _N // _WPB
_GROUPS = _QPW // _L
_BLK = 16


def _ball_query_body(x_hbm, out_hbm, pts_v, out_v):
    c = lax.axis_index("c")
    s = lax.axis_index("s")
    wid = s * _NC + c
    b = wid // _WPB
    qoff = (wid % _WPB) * _QPW

    pltpu.sync_copy(x_hbm.at[b], pts_v)

    lanes = lax.iota(jnp.int32, _L)
    lanes3 = lanes + (lanes + lanes)
    czero = jnp.zeros((_L,), jnp.int32)
    cone = jnp.full((_L,), 1, jnp.int32)
    ctwo = jnp.full((_L,), 2, jnp.int32)
    zero = jnp.zeros((_L,), jnp.int32)

    def group(g, carry_none):
        rows = jnp.full((_L,), g * _L, jnp.int32) + lanes
        qidx = (jnp.full((_L,), qoff, jnp.int32) + rows) * 3
        qx = plsc.load_gather(pts_v, [qidx])
        qy = plsc.load_gather(pts_v, [qidx + 1])
        qz = plsc.load_gather(pts_v, [qidx + 2])

        def cond(carry):
            return carry[0]

        def body(carry):
            j = carry[1]
            st = carry[2:]
            cnt, i0, i1, i2, i3, i4 = st[:6]
            jv0 = jnp.full((_L,), j, dtype=jnp.int32)
            j3 = jv0 + (jv0 + jv0) + lanes3
            xs16 = plsc.load_gather(pts_v, [j3])
            ys16 = plsc.load_gather(pts_v, [j3 + 1])
            zs16 = plsc.load_gather(pts_v, [j3 + 2])
            for u in range(_BLK):
                uvec = jnp.full((_L,), u, jnp.int32)
                px = jnp.take(xs16, uvec)
                py = jnp.take(ys16, uvec)
                pz = jnp.take(zs16, uvec)
                jv = jv0 + u
                dx = qx - px
                dy = qy - py
                dz = qz - pz
                d2 = dx * dx + dy * dy + dz * dz
                m = d2 < _RADIUS2
                i0 = jnp.where(m & (cnt == 0), jv, i0)
                i1 = jnp.where(m & (cnt == 1), jv, i1)
                i2 = jnp.where(m & (cnt == 2), jv, i2)
                i3 = jnp.where(m & (cnt == 3), jv, i3)
                i4 = jnp.where(m & (cnt == 4), jv, i4)
                cnt = cnt + m.astype(jnp.int32)
            jn = j + _BLK
            popc = plsc.all_reduce_population_count(cnt >= _NSAMPLE)
            cont = jnp.logical_and(jn < _N, popc[0] < _L)
            return (cont, jn, cnt, i0, i1, i2, i3, i4)

        init = (jnp.bool_(True), jnp.int32(0), zero, zero, zero, zero, zero,
                zero)
        res = lax.while_loop(cond, body, init)
        cnt, i0, i1, i2, i3, i4 = (res[2], res[3], res[4], res[5], res[6],
                                   res[7])

        o1 = jnp.where(cnt > 1, i1, i0)
        o2 = jnp.where(cnt > 2, i2, i0)
        o3 = jnp.where(cnt > 3, i3, i0)
        o4 = jnp.where(cnt > 4, i4, i0)
        plsc.store_scatter(out_v, [rows, czero], i0)
        plsc.store_scatter(out_v, [rows, cone], o1)
        plsc.store_scatter(out_v, [rows, ctwo], o2)
        plsc.store_scatter(out_v, [rows, ctwo + 1], o3)
        plsc.store_scatter(out_v, [rows, ctwo + 2], o4)
        return carry_none

    lax.fori_loop(0, _GROUPS, group, 0)

    pltpu.sync_copy(out_v, out_hbm.at[b].at[pl.ds(qoff, _QPW)])


def kernel(x):
    mesh = plsc.VectorSubcoreMesh(core_axis_name="c", subcore_axis_name="s")
    return pl.kernel(
        _ball_query_body,
        out_type=jax.ShapeDtypeStruct((_B, _N, _NSAMPLE), jnp.int32),
        mesh=mesh,
        compiler_params=pltpu.CompilerParams(needs_layout_passes=False),
        scratch_types=[
            pltpu.VMEM((_N * 3,), jnp.float32),
            pltpu.VMEM((_QPW, _NSAMPLE), jnp.int32),
        ],
    )(x.reshape(_B, _N * 3))


# speedup vs baseline: 1.1361x; 1.0134x over previous
import jax
import jax.numpy as jnp
from jax import lax
from jax.experimental import pallas as pl
from jax.experimental.pallas import tpu as pltpu
from jax.experimental.pallas import tpu_sc as plsc

_RADIUS2 = 3.4 * 3.4
_NSAMPLE = 5
_B = 8
_N = 2048
_L = 16
_NC = 2
_NS = 16
_NW = _NC * _NS
_WPB = _NW // _B
_QPW = _N // _WPB
_GROUPS = _QPW // _L
_BLK = 4


def _ball_query_body(x_hbm, out_hbm, pts_v, out_v):
    c = lax.axis_index("c")
    s = lax.axis_index("s")
    wid = s * _NC + c
    b = wid // _WPB
    qoff = (wid % _WPB) * _QPW

    pltpu.sync_copy(x_hbm.at[b], pts_v)

    lanes = lax.iota(jnp.int32, _L)
    lanes3 = lanes + (lanes + lanes)
    czero = jnp.zeros((_L,), jnp.int32)
    cone = jnp.full((_L,), 1, jnp.int32)
    ctwo = jnp.full((_L,), 2, jnp.int32)
    zero = jnp.zeros((_L,), jnp.int32)

    def group(g, carry_none):
        rows = jnp.full((_L,), g * _L, jnp.int32) + lanes
        qidx = (jnp.full((_L,), qoff, jnp.int32) + rows) * 3
        qx = plsc.load_gather(pts_v, [qidx])
        qy = plsc.load_gather(pts_v, [qidx + 1])
        qz = plsc.load_gather(pts_v, [qidx + 2])

        def cond(carry):
            return carry[0]

        def body(carry):
            j = carry[1]
            cnt, i0, i1, i2, i3, i4 = carry[2:]
            for u in range(_BLK):
                jv = jnp.full((_L,), j, dtype=jnp.int32) + u
                j3 = jv + (jv + jv)
                px = plsc.load_gather(pts_v, [j3])
                py = plsc.load_gather(pts_v, [j3 + 1])
                pz = plsc.load_gather(pts_v, [j3 + 2])
                dx = qx - px
                dy = qy - py
                dz = qz - pz
                d2 = dx * dx + dy * dy + dz * dz
                m = d2 < _RADIUS2
                i0 = jnp.where(m & (cnt == 0), jv, i0)
                i1 = jnp.where(m & (cnt == 1), jv, i1)
                i2 = jnp.where(m & (cnt == 2), jv, i2)
                i3 = jnp.where(m & (cnt == 3), jv, i3)
                i4 = jnp.where(m & (cnt == 4), jv, i4)
                cnt = cnt + m.astype(jnp.int32)
            jn = j + _BLK
            popc = plsc.all_reduce_population_count(cnt >= _NSAMPLE)
            cont = jnp.logical_and(jn < _N, popc[0] < _L)
            return (cont, jn, cnt, i0, i1, i2, i3, i4)

        init = (jnp.bool_(True), jnp.int32(0), zero, zero, zero, zero, zero,
                zero)
        res = lax.while_loop(cond, body, init)
        cnt, i0, i1, i2, i3, i4 = (res[2], res[3], res[4], res[5], res[6],
                                   res[7])

        o1 = jnp.where(cnt > 1, i1, i0)
        o2 = jnp.where(cnt > 2, i2, i0)
        o3 = jnp.where(cnt > 3, i3, i0)
        o4 = jnp.where(cnt > 4, i4, i0)
        plsc.store_scatter(out_v, [rows, czero], i0)
        plsc.store_scatter(out_v, [rows, cone], o1)
        plsc.store_scatter(out_v, [rows, ctwo], o2)
        plsc.store_scatter(out_v, [rows, ctwo + 1], o3)
        plsc.store_scatter(out_v, [rows, ctwo + 2], o4)
        return carry_none

    lax.fori_loop(0, _GROUPS, group, 0)

    pltpu.sync_copy(out_v, out_hbm.at[b].at[pl.ds(qoff, _QPW)])


def kernel(x):
    mesh = plsc.VectorSubcoreMesh(core_axis_name="c", subcore_axis_name="s")
    return pl.kernel(
        _ball_query_body,
        out_type=jax.ShapeDtypeStruct((_B, _N, _NSAMPLE), jnp.int32),
        mesh=mesh,
        compiler_params=pltpu.CompilerParams(needs_layout_passes=False),
        scratch_types=[
            pltpu.VMEM((_N * 3,), jnp.float32),
            pltpu.VMEM((_QPW, _NSAMPLE), jnp.int32),
        ],
    )(x.reshape(_B, _N * 3))
